# baseline (device time: 326157 ns/iter reference)
import jax
import jax.numpy as jnp
from jax import lax
from jax.experimental import pallas as pl
from jax.experimental.pallas import tpu as pltpu

B, SQ, KV, H, D = 16, 1, 1024, 16, 64
HD = H * D
SCALE = D ** -0.5


def _head_mask(rows, cols_are_hd):
    if cols_are_hd:
        h_idx = lax.broadcasted_iota(jnp.int32, (H, HD), 0)
        j_idx = lax.broadcasted_iota(jnp.int32, (H, HD), 1)
    else:
        j_idx = lax.broadcasted_iota(jnp.int32, (HD, H), 0)
        h_idx = lax.broadcasted_iota(jnp.int32, (HD, H), 1)
    return (j_idx // D == h_idx).astype(jnp.float32)


def kernel(Q, K, V):
    Q2 = Q.reshape(B, HD, 1)

    def body(q_ref, k_ref, v_ref, o_ref,
             ml_send, o_send, ml_recv, o_recv, send_sems, recv_sems):
        b = pl.program_id(0)

        qcol = q_ref[0]
        k2 = k_ref[0].reshape(KV, HD)
        v2 = v_ref[0].reshape(KV, HD)

        qm = (qcol * _head_mask(HD, cols_are_hd=False)).astype(jnp.bfloat16)
        s = lax.dot_general(
            k2.astype(jnp.bfloat16), qm, (((1,), (0,)), ((), ())),
            preferred_element_type=jnp.float32,
        ) * SCALE
        m_b = jnp.max(s, axis=0, keepdims=True)
        p = jnp.exp(s - m_b)
        l_b = jnp.sum(p, axis=0, keepdims=True)
        t = lax.dot_general(
            p.astype(jnp.bfloat16), v2.astype(jnp.bfloat16),
            (((0,), (0,)), ((), ())),
            preferred_element_type=jnp.float32,
        )
        o_b = jnp.sum(t * _head_mask(H, cols_are_hd=True),
                      axis=0, keepdims=True)

        ml_send[pl.ds(b, 1), :] = m_b
        ml_send[pl.ds(B + b, 1), :] = l_b
        o_send[pl.ds(b, 1), :] = o_b

        @pl.when(b == B - 1)
        def _():
            my_x = lax.axis_index("x")
            my_y = lax.axis_index("y")
            my_z = lax.axis_index("z")
            partner = (1 - my_x, my_y, my_z)

            barrier = pltpu.get_barrier_semaphore()
            pl.semaphore_signal(
                barrier, inc=1, device_id=partner,
                device_id_type=pl.DeviceIdType.MESH,
            )
            pl.semaphore_wait(barrier, 1)

            rdma_ml = pltpu.make_async_remote_copy(
                src_ref=ml_send, dst_ref=ml_recv,
                send_sem=send_sems.at[0], recv_sem=recv_sems.at[0],
                device_id=partner, device_id_type=pl.DeviceIdType.MESH,
            )
            rdma_o = pltpu.make_async_remote_copy(
                src_ref=o_send, dst_ref=o_recv,
                send_sem=send_sems.at[1], recv_sem=recv_sems.at[1],
                device_id=partner, device_id_type=pl.DeviceIdType.MESH,
            )
            rdma_ml.start()
            rdma_o.start()
            rdma_ml.wait()
            rdma_o.wait()

            m_l = ml_send[:B, :]
            l_l = ml_send[B:, :]
            m_p = ml_recv[:B, :]
            l_p = ml_recv[B:, :]
            m_n = jnp.maximum(m_l, m_p)
            a_l = jnp.exp(m_l - m_n)
            a_p = jnp.exp(m_p - m_n)
            l_n = a_l * l_l + a_p * l_p
            mask = _head_mask(H, cols_are_hd=True)
            dims = (((1,), (0,)), ((), ()))
            a_l_e = lax.dot_general(a_l, mask, dims,
                                    preferred_element_type=jnp.float32)
            a_p_e = lax.dot_general(a_p, mask, dims,
                                    preferred_element_type=jnp.float32)
            l_n_e = lax.dot_general(l_n, mask, dims,
                                    preferred_element_type=jnp.float32)
            o = (a_l_e * o_send[:, :] + a_p_e * o_recv[:, :]) / l_n_e
            o_ref[:, 0, :] = o

    out = pl.pallas_call(
        body,
        grid=(B,),
        in_specs=[
            pl.BlockSpec((1, HD, 1), lambda b: (b, 0, 0)),
            pl.BlockSpec((1, KV, H, D), lambda b: (b, 0, 0, 0)),
            pl.BlockSpec((1, KV, H, D), lambda b: (b, 0, 0, 0)),
        ],
        out_specs=pl.BlockSpec((B, SQ, HD), lambda b: (0, 0, 0)),
        out_shape=jax.ShapeDtypeStruct((B, SQ, HD), jnp.float32),
        scratch_shapes=[
            pltpu.VMEM((2 * B, H), jnp.float32),
            pltpu.VMEM((B, HD), jnp.float32),
            pltpu.VMEM((2 * B, H), jnp.float32),
            pltpu.VMEM((B, HD), jnp.float32),
            pltpu.SemaphoreType.DMA((2,)),
            pltpu.SemaphoreType.DMA((2,)),
        ],
        compiler_params=pltpu.CompilerParams(
            collective_id=0,
            dimension_semantics=("arbitrary",),
            vmem_limit_bytes=64 * 1024 * 1024,
        ),
    )(Q2, K, V)
    return out.reshape(B, SQ, H, D)


# device time: 192898 ns/iter; 1.6908x vs baseline; 1.6908x over previous
import jax
import jax.numpy as jnp
from jax import lax
from jax.experimental import pallas as pl
from jax.experimental.pallas import tpu as pltpu

B, SQ, KV, H, D = 16, 1, 1024, 16, 64
HD = H * D
SCALE = D ** -0.5


def _head_mask(rows, cols_are_hd):
    if cols_are_hd:
        h_idx = lax.broadcasted_iota(jnp.int32, (H, HD), 0)
        j_idx = lax.broadcasted_iota(jnp.int32, (H, HD), 1)
    else:
        j_idx = lax.broadcasted_iota(jnp.int32, (HD, H), 0)
        h_idx = lax.broadcasted_iota(jnp.int32, (HD, H), 1)
    return (j_idx // D == h_idx).astype(jnp.float32)


def kernel(Q, K, V):
    Q2 = Q.reshape(B, HD, 1)
    K2 = K.reshape(B, KV, HD)
    V2 = V.reshape(B, KV, HD)

    def body(q_ref, k_ref, v_ref, o_ref,
             ml_send, o_send, ml_recv, o_recv, send_sems, recv_sems):
        b = pl.program_id(0)

        qcol = q_ref[0]
        k2 = k_ref[0]
        v2 = v_ref[0]

        qm = (qcol * _head_mask(HD, cols_are_hd=False)).astype(jnp.bfloat16)
        s = lax.dot_general(
            k2.astype(jnp.bfloat16), qm, (((1,), (0,)), ((), ())),
            preferred_element_type=jnp.float32,
        ) * SCALE
        m_b = jnp.max(s, axis=0, keepdims=True)
        p = jnp.exp(s - m_b)
        l_b = jnp.sum(p, axis=0, keepdims=True)
        t = lax.dot_general(
            p.astype(jnp.bfloat16), v2.astype(jnp.bfloat16),
            (((0,), (0,)), ((), ())),
            preferred_element_type=jnp.float32,
        )
        o_b = jnp.sum(t * _head_mask(H, cols_are_hd=True),
                      axis=0, keepdims=True)

        ml_send[pl.ds(b, 1), :] = m_b
        ml_send[pl.ds(B + b, 1), :] = l_b
        o_send[pl.ds(b, 1), :] = o_b

        @pl.when(b == B - 1)
        def _():
            my_x = lax.axis_index("x")
            my_y = lax.axis_index("y")
            my_z = lax.axis_index("z")
            partner = (1 - my_x, my_y, my_z)

            barrier = pltpu.get_barrier_semaphore()
            pl.semaphore_signal(
                barrier, inc=1, device_id=partner,
                device_id_type=pl.DeviceIdType.MESH,
            )
            pl.semaphore_wait(barrier, 1)

            rdma_ml = pltpu.make_async_remote_copy(
                src_ref=ml_send, dst_ref=ml_recv,
                send_sem=send_sems.at[0], recv_sem=recv_sems.at[0],
                device_id=partner, device_id_type=pl.DeviceIdType.MESH,
            )
            rdma_o = pltpu.make_async_remote_copy(
                src_ref=o_send, dst_ref=o_recv,
                send_sem=send_sems.at[1], recv_sem=recv_sems.at[1],
                device_id=partner, device_id_type=pl.DeviceIdType.MESH,
            )
            rdma_ml.start()
            rdma_o.start()
            rdma_ml.wait()
            rdma_o.wait()

            m_l = ml_send[:B, :]
            l_l = ml_send[B:, :]
            m_p = ml_recv[:B, :]
            l_p = ml_recv[B:, :]
            m_n = jnp.maximum(m_l, m_p)
            a_l = jnp.exp(m_l - m_n)
            a_p = jnp.exp(m_p - m_n)
            l_n = a_l * l_l + a_p * l_p
            mask = _head_mask(H, cols_are_hd=True)
            dims = (((1,), (0,)), ((), ()))
            a_l_e = lax.dot_general(a_l, mask, dims,
                                    preferred_element_type=jnp.float32)
            a_p_e = lax.dot_general(a_p, mask, dims,
                                    preferred_element_type=jnp.float32)
            l_n_e = lax.dot_general(l_n, mask, dims,
                                    preferred_element_type=jnp.float32)
            o = (a_l_e * o_send[:, :] + a_p_e * o_recv[:, :]) / l_n_e
            o_ref[:, 0, :] = o

    out = pl.pallas_call(
        body,
        grid=(B,),
        in_specs=[
            pl.BlockSpec((1, HD, 1), lambda b: (b, 0, 0)),
            pl.BlockSpec((1, KV, HD), lambda b: (b, 0, 0)),
            pl.BlockSpec((1, KV, HD), lambda b: (b, 0, 0)),
        ],
        out_specs=pl.BlockSpec((B, SQ, HD), lambda b: (0, 0, 0)),
        out_shape=jax.ShapeDtypeStruct((B, SQ, HD), jnp.float32),
        scratch_shapes=[
            pltpu.VMEM((2 * B, H), jnp.float32),
            pltpu.VMEM((B, HD), jnp.float32),
            pltpu.VMEM((2 * B, H), jnp.float32),
            pltpu.VMEM((B, HD), jnp.float32),
            pltpu.SemaphoreType.DMA((2,)),
            pltpu.SemaphoreType.DMA((2,)),
        ],
        compiler_params=pltpu.CompilerParams(
            collective_id=0,
            dimension_semantics=("arbitrary",),
            vmem_limit_bytes=64 * 1024 * 1024,
        ),
    )(Q2, K2, V2)
    return out.reshape(B, SQ, H, D)
